# baseline (speedup 1.0000x reference)
import functools

import jax
import jax.numpy as jnp
from jax import lax
from jax.experimental import pallas as pl
from jax.experimental.pallas import tpu as pltpu
from jax.experimental.pallas import tpu_sc as plsc

N = 10000
D = 128
NE = 160000
HALF = 5000

NCORES = 2
NSUB = 16
EDGES_PER_TILE = NE // NSUB
CHUNK = 128
NFULL = EDGES_PER_TILE // CHUNK
TAIL = EDGES_PER_TILE - NFULL * CHUNK
RPAD = 5120
RPT = RPAD // NSUB


def _scale_chunk(gb, vv_ref):
    @plsc.parallel_loop(0, CHUNK // 16)
    def scale_body(q):
        vv = vv_ref[pl.ds(q * 16, 16)]
        for j in range(16):
            e = q * 16 + j
            vs = jnp.full((16,), vv[j], jnp.float32)
            for g in range(D // 16):
                sl = pl.ds(g * 16, 16)
                gb[e, sl] = gb[e, sl] * vs


def _sc_spmm_body(ebs_hbm, col_li, row_li, val_li, col_l, row_l, val_l,
                  out_hbm,
                  acc_li, acc_l,
                  colb0, colb1, colb2, rowb0, rowb1, rowb2,
                  valv0, valv1, valv2, gbuf0, gbuf1,
                  ct, rt, valtv, gt_ref,
                  isem0, isem1, isem2, gsem0, gsem1, ssem0, ssem1):
    c = lax.axis_index("c")
    s = lax.axis_index("s")
    colbs = (colb0, colb1, colb2)
    rowbs = (rowb0, rowb1, rowb2)
    valvs = (valv0, valv1, valv2)
    gbufs = (gbuf0, gbuf1)
    isems = (isem0, isem1, isem2)
    gsems = (gsem0, gsem1)
    ssems = (ssem0, ssem1)

    def zero_body(e, _):
        for g in range(D // 16):
            gbuf0[e, pl.ds(g * 16, 16)] = jnp.zeros((16,), jnp.float32)
        return 0
    lax.fori_loop(0, CHUNK, zero_body, 0)
    rs0 = s * RPT
    for off in range(0, RPT, CHUNK):
        span = min(CHUNK, RPT - off)
        pltpu.sync_copy(gbuf0.at[pl.ds(0, span)],
                        acc_li.at[pl.ds(rs0 + off, span)])
        pltpu.sync_copy(gbuf0.at[pl.ds(0, span)],
                        acc_l.at[pl.ds(rs0 + off, span)])
    plsc.subcore_barrier()

    for (colr, rowr, valr, acc) in ((col_li, row_li, val_li, acc_li),
                                    (col_l, row_l, val_l, acc_l)):
        ebase = c * NE + s * EDGES_PER_TILE

        def idx_issue(kc, t):
            b = ebase + kc * CHUNK
            pltpu.async_copy(colr.at[pl.ds(b, CHUNK)], colbs[t], isems[t])
            pltpu.async_copy(rowr.at[pl.ds(b, CHUNK)], rowbs[t], isems[t])
            pltpu.async_copy(valr.at[pl.ds(b, CHUNK)], valvs[t], isems[t])

        def idx_wait(t):
            b = ebase
            pltpu.make_async_copy(colr.at[pl.ds(b, CHUNK)], colbs[t],
                                  isems[t]).wait()
            pltpu.make_async_copy(rowr.at[pl.ds(b, CHUNK)], rowbs[t],
                                  isems[t]).wait()
            pltpu.make_async_copy(valr.at[pl.ds(b, CHUNK)], valvs[t],
                                  isems[t]).wait()

        def step(k, j, guard_first):
            p, q = j % 2, 1 - j % 2
            t, t1, t2 = j % 3, (j + 1) % 3, (j + 2) % 3
            pltpu.make_async_copy(ebs_hbm.at[colbs[t]], gbufs[p],
                                  gsems[p]).wait()
            def wait_prev_scatter():
                pltpu.make_async_copy(gbufs[q], acc.at[rowbs[(j - 1) % 3]],
                                      ssems[q]).wait()
            if guard_first:
                pl.when(k > 0)(wait_prev_scatter)
            else:
                wait_prev_scatter()
            idx_wait(t1)
            pltpu.async_copy(ebs_hbm.at[colbs[t1]], gbufs[q], gsems[q])
            idx_issue(jnp.minimum(k + 2, NFULL - 1), t2)
            _scale_chunk(gbufs[p], valvs[t])
            pltpu.async_copy(gbufs[p], acc.at[rowbs[t]], ssems[p], add=True)

        idx_issue(0, 0)
        idx_issue(1, 1)
        idx_wait(0)
        pltpu.async_copy(ebs_hbm.at[colbs[0]], gbufs[0], gsems[0])

        def body6(i, _):
            for j in range(6):
                step(6 * i + j, j, guard_first=(j == 0))
            return 0
        lax.fori_loop(0, NFULL // 6, body6, 0)

        pltpu.make_async_copy(ebs_hbm.at[colbs[0]], gbufs[0], gsems[0]).wait()
        pltpu.make_async_copy(gbufs[1], acc.at[rowbs[2]], ssems[1]).wait()
        idx_wait(1)

        base = ebase + NFULL * CHUNK
        pltpu.sync_copy(colr.at[pl.ds(base, TAIL)], ct)
        pltpu.sync_copy(rowr.at[pl.ds(base, TAIL)], rt)
        pltpu.sync_copy(valr.at[pl.ds(base, TAIL)], valtv)
        pltpu.async_copy(ebs_hbm.at[ct], gt_ref, gsem0).wait()

        vvt = valtv[...]
        for j in range(TAIL):
            vs = jnp.full((16,), vvt[j], jnp.float32)
            for g in range(D // 16):
                sl = pl.ds(g * 16, 16)
                gt_ref[j, sl] = gt_ref[j, sl] * vs
        pltpu.sync_copy(gt_ref, acc.at[rt], add=True)

    plsc.subcore_barrier()

    rs = s * RPT
    pltpu.sync_copy(acc_li.at[pl.ds(rs, RPT)],
                    out_hbm.at[2 * c].at[pl.ds(rs, RPT)])
    pltpu.sync_copy(acc_l.at[pl.ds(rs, RPT)],
                    out_hbm.at[2 * c + 1].at[pl.ds(rs, RPT)])


def _sc_spmm(ebs, col_li, row_li, val_li, col_l, row_l, val_l):
    mesh = plsc.VectorSubcoreMesh(core_axis_name="c", subcore_axis_name="s")
    return pl.kernel(
        _sc_spmm_body,
        out_type=jax.ShapeDtypeStruct((4, RPAD, D), jnp.float32),
        mesh=mesh,
        scratch_types=(
            [pltpu.VMEM_SHARED((RPAD, D), jnp.float32)] * 2
            + [pltpu.VMEM((CHUNK,), jnp.int32)] * 6
            + [pltpu.VMEM((CHUNK,), jnp.float32)] * 3
            + [pltpu.VMEM((CHUNK, D), jnp.float32)] * 2
            + [pltpu.VMEM((TAIL,), jnp.int32)] * 2
            + [pltpu.VMEM((TAIL,), jnp.float32)]
            + [pltpu.VMEM((TAIL, D), jnp.float32)]
            + [pltpu.SemaphoreType.DMA] * 7
        ),
    )(ebs, col_li, row_li, val_li, col_l, row_l, val_l)


def _tc_dense_body(sc_li, sc_l, e_ref, ws_ref, wd_ref, o_ref):
    li = sc_li[...]
    l = sc_l[...]
    x = jnp.dot(li, ws_ref[...], preferred_element_type=jnp.float32)
    x = x + jnp.dot(l * e_ref[...], wd_ref[...],
                    preferred_element_type=jnp.float32)
    o_ref[...] = jnp.where(x >= 0, x, 0.2 * x)


def _tc_dense(sc_out, ebs, ws, wd):
    RB = 1000
    grid = (2, HALF // RB)
    return pl.pallas_call(
        _tc_dense_body,
        grid=grid,
        in_specs=[
            pl.BlockSpec((None, RB, D), lambda c, j: (2 * c, j, 0)),
            pl.BlockSpec((None, RB, D), lambda c, j: (2 * c + 1, j, 0)),
            pl.BlockSpec((RB, D), lambda c, j: (5 * c + j, 0)),
            pl.BlockSpec((None, D, D), lambda c, j: (c, 0, 0)),
            pl.BlockSpec((None, D, D), lambda c, j: (c, 0, 0)),
        ],
        out_specs=pl.BlockSpec((RB, D), lambda c, j: (5 * c + j, 0)),
        out_shape=jax.ShapeDtypeStruct((N, D), jnp.float32),
    )(sc_out, sc_out, ebs, ws, wd)


def kernel(ebs, li_row_user, li_col_user, li_val_user, l_row_user, l_col_user,
           l_val_user, li_row_item, li_col_item, li_val_item, l_row_item,
           l_col_item, l_val_item, W_side_user, W_dot_user, W_side_item,
           W_dot_item, train_flag):
    col_li = jnp.concatenate([li_col_user, li_col_item]).astype(jnp.int32)
    row_li = jnp.concatenate([li_row_user, li_row_item]).astype(jnp.int32)
    val_li = jnp.concatenate([li_val_user, li_val_item])
    col_l = jnp.concatenate([l_col_user, l_col_item]).astype(jnp.int32)
    row_l = jnp.concatenate([l_row_user, l_row_item]).astype(jnp.int32)
    val_l = jnp.concatenate([l_val_user, l_val_item])
    sc_out = _sc_spmm(ebs, col_li, row_li, val_li, col_l, row_l, val_l)
    ws = jnp.stack([W_side_user, W_side_item])
    wd = jnp.stack([W_dot_user, W_dot_item])
    return _tc_dense(sc_out, ebs, ws, wd)


# speedup vs baseline: 1.1014x; 1.1014x over previous
import functools

import jax
import jax.numpy as jnp
from jax import lax
from jax.experimental import pallas as pl
from jax.experimental.pallas import tpu as pltpu
from jax.experimental.pallas import tpu_sc as plsc

N = 10000
D = 128
NE = 160000
HALF = 5000

NCORES = 2
NSUB = 16
EDGES_PER_TILE = NE // NSUB
CHUNK = 128
NFULL = EDGES_PER_TILE // CHUNK
TAIL = EDGES_PER_TILE - NFULL * CHUNK
RPAD = 5008
RPT = 320
RPT_LAST = RPAD - 15 * RPT


def _scale_chunk(gb, vv_ref, nedges=CHUNK):
    def scale_body(q, _):
        vv = vv_ref[pl.ds(q * 16, 16)]
        for j in range(16):
            e = q * 16 + j
            vs = jnp.full((16,), vv[j], jnp.float32)
            for g in range(D // 16):
                sl = pl.ds(g * 16, 16)
                gb[e, sl] = gb[e, sl] * vs
        return 0
    lax.fori_loop(0, nedges // 16, scale_body, 0)


def _sc_spmm_body(ebs_hbm, col_li, row_li, val_li, col_l, row_l, val_l,
                  out_hbm,
                  acc_li, acc_l,
                  colb0, colb1, colb2, rowb0, rowb1, rowb2,
                  valv0, valv1, valv2, gbuf0, gbuf1, gbuf2,
                  ct, rt, valtv,
                  isem0, isem1, isem2, rsem0, rsem1, rsem2,
                  gsem0, gsem1, gsem2, ssem0, ssem1, ssem2):
    c = lax.axis_index("c")
    s = lax.axis_index("s")
    colbs = (colb0, colb1, colb2)
    rowbs = (rowb0, rowb1, rowb2)
    valvs = (valv0, valv1, valv2)
    gbufs = (gbuf0, gbuf1, gbuf2)
    isems = (isem0, isem1, isem2)
    rsems = (rsem0, rsem1, rsem2)
    gsems = (gsem0, gsem1, gsem2)
    ssems = (ssem0, ssem1, ssem2)

    def zero_body(e, _):
        for g in range(D // 16):
            gbuf0[e, pl.ds(g * 16, 16)] = jnp.zeros((16,), jnp.float32)
        return 0
    lax.fori_loop(0, CHUNK, zero_body, 0)
    rs0 = s * RPT

    @pl.when(s < NSUB - 1)
    def _():
        for off in range(0, RPT, CHUNK):
            span = min(CHUNK, RPT - off)
            pltpu.sync_copy(gbuf0.at[pl.ds(0, span)],
                            acc_li.at[pl.ds(rs0 + off, span)])
            pltpu.sync_copy(gbuf0.at[pl.ds(0, span)],
                            acc_l.at[pl.ds(rs0 + off, span)])

    @pl.when(s == NSUB - 1)
    def _():
        for off in range(0, RPT_LAST, CHUNK):
            span = min(CHUNK, RPT_LAST - off)
            pltpu.sync_copy(gbuf0.at[pl.ds(0, span)],
                            acc_li.at[pl.ds(rs0 + off, span)])
            pltpu.sync_copy(gbuf0.at[pl.ds(0, span)],
                            acc_l.at[pl.ds(rs0 + off, span)])
    plsc.subcore_barrier()

    for (colr, rowr, valr, acc) in ((col_li, row_li, val_li, acc_li),
                                    (col_l, row_l, val_l, acc_l)):
        ebase = c * NE + s * EDGES_PER_TILE

        def cv_issue(kc, t):
            b = ebase + kc * CHUNK
            pltpu.async_copy(colr.at[pl.ds(b, CHUNK)], colbs[t], isems[t])
            pltpu.async_copy(valr.at[pl.ds(b, CHUNK)], valvs[t], isems[t])

        def cv_wait(t):
            b = ebase
            pltpu.make_async_copy(colr.at[pl.ds(b, CHUNK)], colbs[t],
                                  isems[t]).wait()
            pltpu.make_async_copy(valr.at[pl.ds(b, CHUNK)], valvs[t],
                                  isems[t]).wait()

        def row_issue(kc, t):
            b = ebase + kc * CHUNK
            pltpu.async_copy(rowr.at[pl.ds(b, CHUNK)], rowbs[t], rsems[t])

        def row_wait(t):
            pltpu.make_async_copy(rowr.at[pl.ds(ebase, CHUNK)], rowbs[t],
                                  rsems[t]).wait()

        def step(k, j, guard_first):
            u, u1, u2 = j % 3, (j + 1) % 3, (j + 2) % 3
            pltpu.make_async_copy(ebs_hbm.at[colbs[u]], gbufs[u],
                                  gsems[u]).wait()
            cv_wait(u1)
            def wait_prev_scatter():
                pltpu.make_async_copy(gbufs[u1], acc.at[rowbs[u1]],
                                      ssems[u1]).wait()
            if guard_first:
                pl.when(k > 1)(wait_prev_scatter)
            else:
                wait_prev_scatter()
            pltpu.async_copy(ebs_hbm.at[colbs[u1]], gbufs[u1], gsems[u1])
            row_issue(jnp.minimum(k + 1, NFULL - 1), u1)
            cv_issue(jnp.minimum(k + 2, NFULL - 1), u2)
            _scale_chunk(gbufs[u], valvs[u])
            row_wait(u)
            pltpu.async_copy(gbufs[u], acc.at[rowbs[u]], ssems[u], add=True)

        cv_issue(0, 0)
        cv_issue(1, 1)
        row_issue(0, 0)
        cv_wait(0)
        pltpu.async_copy(ebs_hbm.at[colbs[0]], gbufs[0], gsems[0])

        def body6(i, _):
            for j in range(6):
                step(6 * i + j, j, guard_first=(j < 2))
            return 0
        lax.fori_loop(0, NFULL // 6, body6, 0)

        pltpu.make_async_copy(ebs_hbm.at[colbs[0]], gbufs[0], gsems[0]).wait()
        pltpu.make_async_copy(gbufs[1], acc.at[rowbs[1]], ssems[1]).wait()
        pltpu.make_async_copy(gbufs[2], acc.at[rowbs[2]], ssems[2]).wait()
        row_wait(0)
        cv_wait(1)

        base = ebase + NFULL * CHUNK
        pltpu.sync_copy(colr.at[pl.ds(base, TAIL)], ct)
        pltpu.sync_copy(rowr.at[pl.ds(base, TAIL)], rt)
        pltpu.sync_copy(valr.at[pl.ds(base, TAIL)], valtv)
        pltpu.async_copy(ebs_hbm.at[ct], gbufs[0].at[pl.ds(0, TAIL)],
                         gsem0).wait()
        _scale_chunk(gbufs[0], valtv, nedges=TAIL)
        pltpu.sync_copy(gbufs[0].at[pl.ds(0, TAIL)], acc.at[rt], add=True)

    plsc.subcore_barrier()

    rs = s * RPT

    @pl.when(s < NSUB - 1)
    def _():
        pltpu.sync_copy(acc_li.at[pl.ds(rs, RPT)],
                        out_hbm.at[2 * c].at[pl.ds(rs, RPT)])
        pltpu.sync_copy(acc_l.at[pl.ds(rs, RPT)],
                        out_hbm.at[2 * c + 1].at[pl.ds(rs, RPT)])

    @pl.when(s == NSUB - 1)
    def _():
        pltpu.sync_copy(acc_li.at[pl.ds(rs, RPT_LAST)],
                        out_hbm.at[2 * c].at[pl.ds(rs, RPT_LAST)])
        pltpu.sync_copy(acc_l.at[pl.ds(rs, RPT_LAST)],
                        out_hbm.at[2 * c + 1].at[pl.ds(rs, RPT_LAST)])


def _sc_spmm(ebs, col_li, row_li, val_li, col_l, row_l, val_l):
    mesh = plsc.VectorSubcoreMesh(core_axis_name="c", subcore_axis_name="s")
    return pl.kernel(
        _sc_spmm_body,
        out_type=jax.ShapeDtypeStruct((4, RPAD, D), jnp.float32),
        mesh=mesh,
        scratch_types=(
            [pltpu.VMEM_SHARED((RPAD, D), jnp.float32)] * 2
            + [pltpu.VMEM((CHUNK,), jnp.int32)] * 6
            + [pltpu.VMEM((CHUNK,), jnp.float32)] * 3
            + [pltpu.VMEM((CHUNK, D), jnp.float32)] * 3
            + [pltpu.VMEM((TAIL,), jnp.int32)] * 2
            + [pltpu.VMEM((TAIL,), jnp.float32)]
            + [pltpu.SemaphoreType.DMA] * 12
        ),
    )(ebs, col_li, row_li, val_li, col_l, row_l, val_l)


def _tc_dense_body(sc_li, sc_l, e_ref, ws_ref, wd_ref, o_ref):
    li = sc_li[...]
    l = sc_l[...]
    x = jnp.dot(li, ws_ref[...], preferred_element_type=jnp.float32)
    x = x + jnp.dot(l * e_ref[...], wd_ref[...],
                    preferred_element_type=jnp.float32)
    o_ref[...] = jnp.where(x >= 0, x, 0.2 * x)


def _tc_dense(sc_out, ebs, ws, wd):
    RB = 1000
    grid = (2, HALF // RB)
    return pl.pallas_call(
        _tc_dense_body,
        grid=grid,
        in_specs=[
            pl.BlockSpec((None, RB, D), lambda c, j: (2 * c, j, 0)),
            pl.BlockSpec((None, RB, D), lambda c, j: (2 * c + 1, j, 0)),
            pl.BlockSpec((RB, D), lambda c, j: (5 * c + j, 0)),
            pl.BlockSpec((None, D, D), lambda c, j: (c, 0, 0)),
            pl.BlockSpec((None, D, D), lambda c, j: (c, 0, 0)),
        ],
        out_specs=pl.BlockSpec((RB, D), lambda c, j: (5 * c + j, 0)),
        out_shape=jax.ShapeDtypeStruct((N, D), jnp.float32),
    )(sc_out, sc_out, ebs, ws, wd)


def kernel(ebs, li_row_user, li_col_user, li_val_user, l_row_user, l_col_user,
           l_val_user, li_row_item, li_col_item, li_val_item, l_row_item,
           l_col_item, l_val_item, W_side_user, W_dot_user, W_side_item,
           W_dot_item, train_flag):
    col_li = jnp.concatenate([li_col_user, li_col_item]).astype(jnp.int32)
    row_li = jnp.concatenate([li_row_user, li_row_item]).astype(jnp.int32)
    val_li = jnp.concatenate([li_val_user, li_val_item])
    col_l = jnp.concatenate([l_col_user, l_col_item]).astype(jnp.int32)
    row_l = jnp.concatenate([l_row_user, l_row_item]).astype(jnp.int32)
    val_l = jnp.concatenate([l_val_user, l_val_item])
    sc_out = _sc_spmm(ebs, col_li, row_li, val_li, col_l, row_l, val_l)
    ws = jnp.stack([W_side_user, W_side_item])
    wd = jnp.stack([W_dot_user, W_dot_item])
    return _tc_dense(sc_out, ebs, ws, wd)


# speedup vs baseline: 1.1084x; 1.0064x over previous
import functools

import jax
import jax.numpy as jnp
from jax import lax
from jax.experimental import pallas as pl
from jax.experimental.pallas import tpu as pltpu
from jax.experimental.pallas import tpu_sc as plsc

N = 10000
D = 128
NE = 160000
HALF = 5000

NCORES = 2
NSUB = 16
EDGES_PER_TILE = NE // NSUB
CHUNK = 128
NFULL = EDGES_PER_TILE // CHUNK
TAIL = EDGES_PER_TILE - NFULL * CHUNK
RPAD = 5008
RPT = 320
RPT_LAST = RPAD - 15 * RPT


def _scale_chunk(gb, vv_ref, lo=0, nedges=CHUNK):
    def scale_body(q, _):
        vv = vv_ref[pl.ds(lo + q * 16, 16)]
        for j in range(16):
            e = lo + q * 16 + j
            vs = jnp.full((16,), vv[j], jnp.float32)
            for g in range(D // 16):
                sl = pl.ds(g * 16, 16)
                gb[e, sl] = gb[e, sl] * vs
        return 0
    lax.fori_loop(0, nedges // 16, scale_body, 0)


def _sc_spmm_body(ebs_hbm, col_li, row_li, val_li, col_l, row_l, val_l,
                  out_hbm,
                  acc_li, acc_l,
                  colb0, colb1, colb2, rowb0, rowb1, rowb2,
                  valv0, valv1, valv2, gbuf0, gbuf1, gbuf2,
                  ct, rt, valtv,
                  isem0, isem1, isem2, rsem0, rsem1, rsem2,
                  gsem0, gsem1, gsem2, hsem0, hsem1, hsem2,
                  ssem0, ssem1, ssem2):
    c = lax.axis_index("c")
    s = lax.axis_index("s")
    colbs = (colb0, colb1, colb2)
    rowbs = (rowb0, rowb1, rowb2)
    valvs = (valv0, valv1, valv2)
    gbufs = (gbuf0, gbuf1, gbuf2)
    isems = (isem0, isem1, isem2)
    rsems = (rsem0, rsem1, rsem2)
    gsems = (gsem0, gsem1, gsem2)
    hsems = (hsem0, hsem1, hsem2)
    ssems = (ssem0, ssem1, ssem2)

    def zero_body(e, _):
        for g in range(D // 16):
            gbuf0[e, pl.ds(g * 16, 16)] = jnp.zeros((16,), jnp.float32)
        return 0
    lax.fori_loop(0, CHUNK, zero_body, 0)
    rs0 = s * RPT

    @pl.when(s < NSUB - 1)
    def _():
        for off in range(0, RPT, CHUNK):
            span = min(CHUNK, RPT - off)
            pltpu.sync_copy(gbuf0.at[pl.ds(0, span)],
                            acc_li.at[pl.ds(rs0 + off, span)])
            pltpu.sync_copy(gbuf0.at[pl.ds(0, span)],
                            acc_l.at[pl.ds(rs0 + off, span)])

    @pl.when(s == NSUB - 1)
    def _():
        for off in range(0, RPT_LAST, CHUNK):
            span = min(CHUNK, RPT_LAST - off)
            pltpu.sync_copy(gbuf0.at[pl.ds(0, span)],
                            acc_li.at[pl.ds(rs0 + off, span)])
            pltpu.sync_copy(gbuf0.at[pl.ds(0, span)],
                            acc_l.at[pl.ds(rs0 + off, span)])
    plsc.subcore_barrier()

    for (colr, rowr, valr, acc) in ((col_li, row_li, val_li, acc_li),
                                    (col_l, row_l, val_l, acc_l)):
        ebase = c * NE + s * EDGES_PER_TILE

        def cv_issue(kc, t):
            b = ebase + kc * CHUNK
            pltpu.async_copy(colr.at[pl.ds(b, CHUNK)], colbs[t], isems[t])
            pltpu.async_copy(valr.at[pl.ds(b, CHUNK)], valvs[t], isems[t])

        def cv_wait(t):
            b = ebase
            pltpu.make_async_copy(colr.at[pl.ds(b, CHUNK)], colbs[t],
                                  isems[t]).wait()
            pltpu.make_async_copy(valr.at[pl.ds(b, CHUNK)], valvs[t],
                                  isems[t]).wait()

        def row_issue(kc, t):
            b = ebase + kc * CHUNK
            pltpu.async_copy(rowr.at[pl.ds(b, CHUNK)], rowbs[t], rsems[t])

        def row_wait(t):
            pltpu.make_async_copy(rowr.at[pl.ds(ebase, CHUNK)], rowbs[t],
                                  rsems[t]).wait()

        HC = CHUNK // 2

        def gather_issue(t):
            pltpu.async_copy(ebs_hbm.at[colbs[t].at[pl.ds(0, HC)]],
                             gbufs[t].at[pl.ds(0, HC)], gsems[t])
            pltpu.async_copy(ebs_hbm.at[colbs[t].at[pl.ds(HC, HC)]],
                             gbufs[t].at[pl.ds(HC, HC)], hsems[t])

        def gather_wait_a(t):
            pltpu.make_async_copy(ebs_hbm.at[colbs[t].at[pl.ds(0, HC)]],
                                  gbufs[t].at[pl.ds(0, HC)], gsems[t]).wait()

        def gather_wait_b(t):
            pltpu.make_async_copy(ebs_hbm.at[colbs[t].at[pl.ds(HC, HC)]],
                                  gbufs[t].at[pl.ds(HC, HC)], hsems[t]).wait()

        def step(k, j, guard_first):
            u, u1, u2 = j % 3, (j + 1) % 3, (j + 2) % 3
            gather_wait_a(u)
            cv_wait(u1)
            def wait_prev_scatter():
                pltpu.make_async_copy(gbufs[u1], acc.at[rowbs[u1]],
                                      ssems[u1]).wait()
            if guard_first:
                pl.when(k > 1)(wait_prev_scatter)
            else:
                wait_prev_scatter()
            gather_issue(u1)
            row_issue(jnp.minimum(k + 1, NFULL - 1), u1)
            cv_issue(jnp.minimum(k + 2, NFULL - 1), u2)
            _scale_chunk(gbufs[u], valvs[u], 0, HC)
            gather_wait_b(u)
            _scale_chunk(gbufs[u], valvs[u], HC, HC)
            row_wait(u)
            pltpu.async_copy(gbufs[u], acc.at[rowbs[u]], ssems[u], add=True)

        cv_issue(0, 0)
        cv_issue(1, 1)
        row_issue(0, 0)
        cv_wait(0)
        gather_issue(0)

        def body6(i, _):
            for j in range(6):
                step(6 * i + j, j, guard_first=(j < 2))
            return 0
        lax.fori_loop(0, NFULL // 6, body6, 0)

        gather_wait_a(0)
        gather_wait_b(0)
        pltpu.make_async_copy(gbufs[1], acc.at[rowbs[1]], ssems[1]).wait()
        pltpu.make_async_copy(gbufs[2], acc.at[rowbs[2]], ssems[2]).wait()
        row_wait(0)
        cv_wait(1)

        base = ebase + NFULL * CHUNK
        pltpu.sync_copy(colr.at[pl.ds(base, TAIL)], ct)
        pltpu.sync_copy(rowr.at[pl.ds(base, TAIL)], rt)
        pltpu.sync_copy(valr.at[pl.ds(base, TAIL)], valtv)
        pltpu.async_copy(ebs_hbm.at[ct], gbufs[0].at[pl.ds(0, TAIL)],
                         gsem0).wait()
        _scale_chunk(gbufs[0], valtv, nedges=TAIL)
        pltpu.sync_copy(gbufs[0].at[pl.ds(0, TAIL)], acc.at[rt], add=True)

    plsc.subcore_barrier()

    rs = s * RPT

    @pl.when(s < NSUB - 1)
    def _():
        pltpu.sync_copy(acc_li.at[pl.ds(rs, RPT)],
                        out_hbm.at[2 * c].at[pl.ds(rs, RPT)])
        pltpu.sync_copy(acc_l.at[pl.ds(rs, RPT)],
                        out_hbm.at[2 * c + 1].at[pl.ds(rs, RPT)])

    @pl.when(s == NSUB - 1)
    def _():
        pltpu.sync_copy(acc_li.at[pl.ds(rs, RPT_LAST)],
                        out_hbm.at[2 * c].at[pl.ds(rs, RPT_LAST)])
        pltpu.sync_copy(acc_l.at[pl.ds(rs, RPT_LAST)],
                        out_hbm.at[2 * c + 1].at[pl.ds(rs, RPT_LAST)])


def _sc_spmm(ebs, col_li, row_li, val_li, col_l, row_l, val_l):
    mesh = plsc.VectorSubcoreMesh(core_axis_name="c", subcore_axis_name="s")
    return pl.kernel(
        _sc_spmm_body,
        out_type=jax.ShapeDtypeStruct((4, RPAD, D), jnp.float32),
        mesh=mesh,
        scratch_types=(
            [pltpu.VMEM_SHARED((RPAD, D), jnp.float32)] * 2
            + [pltpu.VMEM((CHUNK,), jnp.int32)] * 6
            + [pltpu.VMEM((CHUNK,), jnp.float32)] * 3
            + [pltpu.VMEM((CHUNK, D), jnp.float32)] * 3
            + [pltpu.VMEM((TAIL,), jnp.int32)] * 2
            + [pltpu.VMEM((TAIL,), jnp.float32)]
            + [pltpu.SemaphoreType.DMA] * 15
        ),
    )(ebs, col_li, row_li, val_li, col_l, row_l, val_l)


def _tc_dense_body(sc_li, sc_l, e_ref, ws_ref, wd_ref, o_ref):
    li = sc_li[...]
    l = sc_l[...]
    x = jnp.dot(li, ws_ref[...], preferred_element_type=jnp.float32)
    x = x + jnp.dot(l * e_ref[...], wd_ref[...],
                    preferred_element_type=jnp.float32)
    o_ref[...] = jnp.where(x >= 0, x, 0.2 * x)


def _tc_dense(sc_out, ebs, ws, wd):
    RB = 1000
    grid = (2, HALF // RB)
    return pl.pallas_call(
        _tc_dense_body,
        grid=grid,
        in_specs=[
            pl.BlockSpec((None, RB, D), lambda c, j: (2 * c, j, 0)),
            pl.BlockSpec((None, RB, D), lambda c, j: (2 * c + 1, j, 0)),
            pl.BlockSpec((RB, D), lambda c, j: (5 * c + j, 0)),
            pl.BlockSpec((None, D, D), lambda c, j: (c, 0, 0)),
            pl.BlockSpec((None, D, D), lambda c, j: (c, 0, 0)),
        ],
        out_specs=pl.BlockSpec((RB, D), lambda c, j: (5 * c + j, 0)),
        out_shape=jax.ShapeDtypeStruct((N, D), jnp.float32),
    )(sc_out, sc_out, ebs, ws, wd)


def kernel(ebs, li_row_user, li_col_user, li_val_user, l_row_user, l_col_user,
           l_val_user, li_row_item, li_col_item, li_val_item, l_row_item,
           l_col_item, l_val_item, W_side_user, W_dot_user, W_side_item,
           W_dot_item, train_flag):
    col_li = jnp.concatenate([li_col_user, li_col_item]).astype(jnp.int32)
    row_li = jnp.concatenate([li_row_user, li_row_item]).astype(jnp.int32)
    val_li = jnp.concatenate([li_val_user, li_val_item])
    col_l = jnp.concatenate([l_col_user, l_col_item]).astype(jnp.int32)
    row_l = jnp.concatenate([l_row_user, l_row_item]).astype(jnp.int32)
    val_l = jnp.concatenate([l_val_user, l_val_item])
    sc_out = _sc_spmm(ebs, col_li, row_li, val_li, col_l, row_l, val_l)
    ws = jnp.stack([W_side_user, W_side_item])
    wd = jnp.stack([W_dot_user, W_dot_item])
    return _tc_dense(sc_out, ebs, ws, wd)


# speedup vs baseline: 1.1087x; 1.0002x over previous
import jax
import jax.numpy as jnp
from jax import lax
from jax.experimental import pallas as pl
from jax.experimental.pallas import tpu as pltpu
from jax.experimental.pallas import tpu_sc as plsc

N = 10000
D = 128
NE = 160000
HALF = 5000

NCORES = 2
NSUB = 16
EDGES_PER_TILE = NE // NSUB
CHUNK = 128
NFULL = EDGES_PER_TILE // CHUNK
TAIL = EDGES_PER_TILE - NFULL * CHUNK
RPAD = 5008
RPT = 320
RPT_LAST = RPAD - 15 * RPT


def _scale_chunk(gb, vv_ref, lo=0, nedges=CHUNK):
    def scale_body(q, _):
        vv = vv_ref[pl.ds(lo + q * 16, 16)]
        for j in range(16):
            e = lo + q * 16 + j
            vs = jnp.full((16,), vv[j], jnp.float32)
            for g in range(D // 16):
                sl = pl.ds(g * 16, 16)
                gb[e, sl] = gb[e, sl] * vs
        return 0
    lax.fori_loop(0, nedges // 16, scale_body, 0)


def _sc_spmm_body(ebs_hbm, col_li, row_li, val_li, col_l, row_l, val_l,
                  out_hbm,
                  acc_li, acc_l,
                  colb0, colb1, colb2, rowb0, rowb1, rowb2,
                  valv0, valv1, valv2, gbuf0, gbuf1, gbuf2,
                  ct, rt, valtv,
                  isem0, isem1, isem2, rsem0, rsem1, rsem2,
                  gsem0, gsem1, gsem2, hsem0, hsem1, hsem2,
                  ssem0, ssem1, ssem2):
    c = lax.axis_index("c")
    s = lax.axis_index("s")
    colbs = (colb0, colb1, colb2)
    rowbs = (rowb0, rowb1, rowb2)
    valvs = (valv0, valv1, valv2)
    gbufs = (gbuf0, gbuf1, gbuf2)
    isems = (isem0, isem1, isem2)
    rsems = (rsem0, rsem1, rsem2)
    gsems = (gsem0, gsem1, gsem2)
    hsems = (hsem0, hsem1, hsem2)
    ssems = (ssem0, ssem1, ssem2)

    def zero_body(e, _):
        for g in range(D // 16):
            gbuf0[e, pl.ds(g * 16, 16)] = jnp.zeros((16,), jnp.float32)
        return 0
    lax.fori_loop(0, CHUNK, zero_body, 0)
    rs0 = s * RPT

    @pl.when(s < NSUB - 1)
    def _():
        for off in range(0, RPT, CHUNK):
            span = min(CHUNK, RPT - off)
            pltpu.sync_copy(gbuf0.at[pl.ds(0, span)],
                            acc_li.at[pl.ds(rs0 + off, span)])
            pltpu.sync_copy(gbuf0.at[pl.ds(0, span)],
                            acc_l.at[pl.ds(rs0 + off, span)])

    @pl.when(s == NSUB - 1)
    def _():
        for off in range(0, RPT_LAST, CHUNK):
            span = min(CHUNK, RPT_LAST - off)
            pltpu.sync_copy(gbuf0.at[pl.ds(0, span)],
                            acc_li.at[pl.ds(rs0 + off, span)])
            pltpu.sync_copy(gbuf0.at[pl.ds(0, span)],
                            acc_l.at[pl.ds(rs0 + off, span)])
    plsc.subcore_barrier()

    for (colr, rowr, valr, acc) in ((col_li, row_li, val_li, acc_li),
                                    (col_l, row_l, val_l, acc_l)):
        ebase = c * NE + s * EDGES_PER_TILE

        def cv_issue(kc, t):
            b = ebase + kc * CHUNK
            pltpu.async_copy(colr.at[pl.ds(b, CHUNK)], colbs[t], isems[t])
            pltpu.async_copy(valr.at[pl.ds(b, CHUNK)], valvs[t], isems[t])

        def cv_wait(t):
            b = ebase
            pltpu.make_async_copy(colr.at[pl.ds(b, CHUNK)], colbs[t],
                                  isems[t]).wait()
            pltpu.make_async_copy(valr.at[pl.ds(b, CHUNK)], valvs[t],
                                  isems[t]).wait()

        def row_issue(kc, t):
            b = ebase + kc * CHUNK
            pltpu.async_copy(rowr.at[pl.ds(b, CHUNK)], rowbs[t], rsems[t])

        def row_wait(t):
            pltpu.make_async_copy(rowr.at[pl.ds(ebase, CHUNK)], rowbs[t],
                                  rsems[t]).wait()

        HC = CHUNK // 2

        def gather_issue(t):
            pltpu.async_copy(ebs_hbm.at[colbs[t].at[pl.ds(0, HC)]],
                             gbufs[t].at[pl.ds(0, HC)], gsems[t])
            pltpu.async_copy(ebs_hbm.at[colbs[t].at[pl.ds(HC, HC)]],
                             gbufs[t].at[pl.ds(HC, HC)], hsems[t])

        def gather_wait_a(t):
            pltpu.make_async_copy(ebs_hbm.at[colbs[t].at[pl.ds(0, HC)]],
                                  gbufs[t].at[pl.ds(0, HC)], gsems[t]).wait()

        def gather_wait_b(t):
            pltpu.make_async_copy(ebs_hbm.at[colbs[t].at[pl.ds(HC, HC)]],
                                  gbufs[t].at[pl.ds(HC, HC)], hsems[t]).wait()

        def step(k, j, guard_first):
            u, u1, u2 = j % 3, (j + 1) % 3, (j + 2) % 3
            gather_wait_a(u)
            cv_wait(u1)
            def wait_prev_scatter():
                pltpu.make_async_copy(gbufs[u1], acc.at[rowbs[u1]],
                                      ssems[u1]).wait()
            if guard_first:
                pl.when(k > 1)(wait_prev_scatter)
            else:
                wait_prev_scatter()
            gather_issue(u1)
            row_issue(jnp.minimum(k + 1, NFULL - 1), u1)
            cv_issue(jnp.minimum(k + 2, NFULL - 1), u2)
            _scale_chunk(gbufs[u], valvs[u], 0, HC)
            gather_wait_b(u)
            _scale_chunk(gbufs[u], valvs[u], HC, HC)
            row_wait(u)
            pltpu.async_copy(gbufs[u], acc.at[rowbs[u]], ssems[u], add=True)

        cv_issue(0, 0)
        cv_issue(1, 1)
        row_issue(0, 0)
        cv_wait(0)
        gather_issue(0)

        def body6(i, _):
            for j in range(6):
                step(6 * i + j, j, guard_first=(j < 2))
            return 0
        lax.fori_loop(0, NFULL // 6, body6, 0)

        gather_wait_a(0)
        gather_wait_b(0)
        pltpu.make_async_copy(gbufs[1], acc.at[rowbs[1]], ssems[1]).wait()
        pltpu.make_async_copy(gbufs[2], acc.at[rowbs[2]], ssems[2]).wait()
        row_wait(0)
        cv_wait(1)

        base = ebase + NFULL * CHUNK
        pltpu.sync_copy(colr.at[pl.ds(base, TAIL)], ct)
        pltpu.sync_copy(rowr.at[pl.ds(base, TAIL)], rt)
        pltpu.sync_copy(valr.at[pl.ds(base, TAIL)], valtv)
        pltpu.async_copy(ebs_hbm.at[ct], gbufs[0].at[pl.ds(0, TAIL)],
                         gsem0).wait()
        _scale_chunk(gbufs[0], valtv, nedges=TAIL)
        pltpu.sync_copy(gbufs[0].at[pl.ds(0, TAIL)], acc.at[rt], add=True)

    plsc.subcore_barrier()

    rs = s * RPT

    @pl.when(s < NSUB - 1)
    def _():
        pltpu.sync_copy(acc_li.at[pl.ds(rs, RPT)],
                        out_hbm.at[2 * c].at[pl.ds(rs, RPT)])
        pltpu.sync_copy(acc_l.at[pl.ds(rs, RPT)],
                        out_hbm.at[2 * c + 1].at[pl.ds(rs, RPT)])

    @pl.when(s == NSUB - 1)
    def _():
        pltpu.sync_copy(acc_li.at[pl.ds(rs, RPT_LAST)],
                        out_hbm.at[2 * c].at[pl.ds(rs, RPT_LAST)])
        pltpu.sync_copy(acc_l.at[pl.ds(rs, RPT_LAST)],
                        out_hbm.at[2 * c + 1].at[pl.ds(rs, RPT_LAST)])


def _sc_spmm(ebs, col_li, row_li, val_li, col_l, row_l, val_l):
    mesh = plsc.VectorSubcoreMesh(core_axis_name="c", subcore_axis_name="s")
    return pl.kernel(
        _sc_spmm_body,
        out_type=jax.ShapeDtypeStruct((4, RPAD, D), jnp.float32),
        mesh=mesh,
        scratch_types=(
            [pltpu.VMEM_SHARED((RPAD, D), jnp.float32)] * 2
            + [pltpu.VMEM((CHUNK,), jnp.int32)] * 6
            + [pltpu.VMEM((CHUNK,), jnp.float32)] * 3
            + [pltpu.VMEM((CHUNK, D), jnp.float32)] * 3
            + [pltpu.VMEM((TAIL,), jnp.int32)] * 2
            + [pltpu.VMEM((TAIL,), jnp.float32)]
            + [pltpu.SemaphoreType.DMA] * 15
        ),
    )(ebs, col_li, row_li, val_li, col_l, row_l, val_l)


def _tc_dense_body(sc_li, sc_l, e_ref, ws_ref, wd_ref, o_ref):
    li = sc_li[...]
    l = sc_l[...]
    x = jnp.dot(li, ws_ref[...], preferred_element_type=jnp.float32)
    x = x + jnp.dot(l * e_ref[...], wd_ref[...],
                    preferred_element_type=jnp.float32)
    o_ref[...] = jnp.where(x >= 0, x, 0.2 * x)


def _tc_dense(sc_out, ebs, ws, wd):
    RB = 1000
    grid = (2, HALF // RB)
    return pl.pallas_call(
        _tc_dense_body,
        grid=grid,
        in_specs=[
            pl.BlockSpec((None, RB, D), lambda c, j: (2 * c, j, 0)),
            pl.BlockSpec((None, RB, D), lambda c, j: (2 * c + 1, j, 0)),
            pl.BlockSpec((RB, D), lambda c, j: (5 * c + j, 0)),
            pl.BlockSpec((None, D, D), lambda c, j: (c, 0, 0)),
            pl.BlockSpec((None, D, D), lambda c, j: (c, 0, 0)),
        ],
        out_specs=pl.BlockSpec((RB, D), lambda c, j: (5 * c + j, 0)),
        out_shape=jax.ShapeDtypeStruct((N, D), jnp.float32),
    )(sc_out, sc_out, ebs, ws, wd)


def kernel(ebs, li_row_user, li_col_user, li_val_user, l_row_user, l_col_user,
           l_val_user, li_row_item, li_col_item, li_val_item, l_row_item,
           l_col_item, l_val_item, W_side_user, W_dot_user, W_side_item,
           W_dot_item, train_flag):
    col_li = jnp.concatenate([li_col_user, li_col_item]).astype(jnp.int32)
    row_li = jnp.concatenate([li_row_user, li_row_item]).astype(jnp.int32)
    val_li = jnp.concatenate([li_val_user, li_val_item])
    col_l = jnp.concatenate([l_col_user, l_col_item]).astype(jnp.int32)
    row_l = jnp.concatenate([l_row_user, l_row_item]).astype(jnp.int32)
    val_l = jnp.concatenate([l_val_user, l_val_item])
    sc_out = _sc_spmm(ebs, col_li, row_li, val_li, col_l, row_l, val_l)
    ws = jnp.stack([W_side_user, W_side_item])
    wd = jnp.stack([W_dot_user, W_dot_item])
    return _tc_dense(sc_out, ebs, ws, wd)
